# baseline (speedup 1.0000x reference)
import functools

import jax
import jax.numpy as jnp
from jax import lax
from jax.experimental import pallas as pl
from jax.experimental.pallas import tpu as pltpu
from jax.experimental.pallas import tpu_sc as plsc

_B, _S, _H = 1024, 50, 512
_NW = 32
_ROWS_PER_W = _B // _NW
_CHUNK_ROWS = 1
_NCHUNK = _ROWS_PER_W // _CHUNK_ROWS
_NBUF = 4
_K = 2


def _build_sc_copy():
    mesh = plsc.VectorSubcoreMesh(core_axis_name="c", subcore_axis_name="s")

    @functools.partial(
        pl.kernel,
        mesh=mesh,
        out_type=jax.ShapeDtypeStruct((_B, _S, _H), jnp.float32),
        scratch_types=(
            [pltpu.VMEM((_CHUNK_ROWS, _S, _H), jnp.float32)
             for _ in range(_NBUF)]
            + [pltpu.SemaphoreType.DMA for _ in range(2 * _NBUF)]
        ),
        compiler_params=pltpu.CompilerParams(use_tc_tiling_on_sc=True),
    )
    def sc_copy(x_hbm, o_hbm, *scratch):
        bufs = scratch[:_NBUF]
        isems = scratch[_NBUF:2 * _NBUF]
        osems = scratch[2 * _NBUF:]
        wid = lax.axis_index("s") * 2 + lax.axis_index("c")
        base = wid * _ROWS_PER_W

        def in_copy(i):
            s = i % _NBUF
            return pltpu.make_async_copy(
                x_hbm.at[pl.ds(base + i * _CHUNK_ROWS, _CHUNK_ROWS)],
                bufs[s], isems[s])

        def out_copy(i):
            s = i % _NBUF
            return pltpu.make_async_copy(
                bufs[s],
                o_hbm.at[pl.ds(base + i * _CHUNK_ROWS, _CHUNK_ROWS)],
                osems[s])

        waited_outs = set()
        for j in range(min(_K, _NCHUNK)):
            in_copy(j).start()
        for i in range(_NCHUNK):
            j = i + _K
            if j < _NCHUNK:
                if j - _NBUF >= 0:
                    out_copy(j - _NBUF).wait()
                    waited_outs.add(j - _NBUF)
                in_copy(j).start()
            in_copy(i).wait()
            out_copy(i).start()
        for i in range(_NCHUNK):
            if i not in waited_outs:
                out_copy(i).wait()

    return sc_copy


_sc_copy = _build_sc_copy()


def kernel(x, hippocampus, neocortex):
    return _sc_copy(x)
